# baseline (speedup 1.0000x reference)
import functools

import numpy as np
import jax
import jax.numpy as jnp
from jax import lax
from jax.experimental import pallas as pl
from jax.experimental.pallas import tpu as pltpu
from jax.experimental.pallas import tpu_sc as plsc

_N, _T, _D = 128, 2048, 128


def _crop_constants():
    np.random.seed(1)
    crop_l = int(np.random.randint(low=2, high=_T + 1))
    crop_left = int(np.random.randint(_T - crop_l + 1))
    crop_right = crop_left + crop_l
    crop_eleft = int(np.random.randint(crop_left + 1))
    crop_eright = int(np.random.randint(low=crop_right, high=_T + 1))
    crop_offset = np.random.randint(
        low=-crop_eleft, high=_T - crop_eright + 1, size=_N
    )
    return crop_l, crop_left, crop_eleft, crop_offset


_CROP_L, _CROP_LEFT, _CROP_ELEFT, _OFF = _crop_constants()
_START = _OFF + _CROP_LEFT
_B = _N * _CROP_L

_NC, _NS = 2, 16
_NW = _NC * _NS
_RPW = _N // _NW
_CB = 448

_BASES = np.zeros((_NW, _RPW), np.int64)
for _w in range(_NW):
    for _j in range(_RPW):
        _r = _w * _RPW + _j
        _BASES[_w, _j] = _r * _T + _START[_r]

_CHUNKS = []
for _j in range(_RPW):
    _o = 0
    while _o < _CROP_L:
        _ln = min(_CB, _CROP_L - _o)
        if _ln % 8:
            _ln8 = -(-_ln // 8) * 8
            _CHUNKS.append((_j, _CROP_L - _ln8, _ln8))
            break
        _CHUNKS.append((_j, _o, _ln))
        _o += _ln


def _crop_copy(x2d):
    mesh = plsc.VectorSubcoreMesh(core_axis_name="c", subcore_axis_name="s")
    out_sds = jax.ShapeDtypeStruct((_CROP_L, _N, _D), jnp.float32)

    @functools.partial(
        pl.kernel,
        mesh=mesh,
        out_type=out_sds,
        compiler_params=pltpu.CompilerParams(use_tc_tiling_on_sc=False),
        scratch_types=[
            pltpu.VMEM((_CB, _D), jnp.float32),
            pltpu.VMEM((_CB, _D), jnp.float32),
            pltpu.SemaphoreType.DMA,
            pltpu.SemaphoreType.DMA,
            pltpu.SemaphoreType.DMA,
            pltpu.SemaphoreType.DMA,
        ],
    )
    def k(x_hbm, out_a, buf0, buf1, sem0, sem1, wa0, wa1):
        wid = lax.axis_index("s") * _NC + lax.axis_index("c")
        src_base = []
        for j in range(_RPW):
            b = jnp.int32(int(_BASES[0, j]))
            for w in range(1, _NW):
                b = jnp.where(wid == w, jnp.int32(int(_BASES[w, j])), b)
            src_base.append(b)
        bufs = (buf0, buf1)
        sems = (sem0, sem1)
        wsems_a = (wa0, wa1)

        def src(t):
            j, o, ln = _CHUNKS[t]
            off = src_base[j] + o
            return x_hbm.at[pl.ds(off, ln)]

        def stage(t):
            j, o, ln = _CHUNKS[t]
            return bufs[t % 2].at[pl.ds(0, ln)]

        def dst(t, out_hbm):
            j, o, ln = _CHUNKS[t]
            rb = wid * _RPW + j
            return out_hbm.at[pl.ds(o, ln), rb]

        def wcopy_a(t):
            return pltpu.make_async_copy(
                stage(t), dst(t, out_a), wsems_a[t % 2]
            )

        n = len(_CHUNKS)
        pltpu.make_async_copy(src(0), stage(0), sems[0]).start()
        for t in range(n):
            b = t % 2
            pltpu.make_async_copy(src(t), stage(t), sems[b]).wait()
            if t + 1 < n:
                if t >= 1:
                    wcopy_a(t - 1).wait()
                pltpu.make_async_copy(
                    src(t + 1), stage(t + 1), sems[1 - b]
                ).start()
            wcopy_a(t).start()
        if n >= 2:
            wcopy_a(n - 2).wait()
        wcopy_a(n - 1).wait()

    return k(x2d)


_NSEM_TC = 8


def _dup_tc(x2d):
    def tck(x_hbm, out_hbm, *sems):
        def cp(i):
            base = i * _T + int(_START[i])
            return pltpu.make_async_copy(
                x_hbm.at[pl.ds(base, _CROP_L)],
                out_hbm.at[:, i],
                sems[i % _NSEM_TC],
            )

        for i in range(_N):
            if i >= _NSEM_TC:
                cp(i - _NSEM_TC).wait()
            cp(i).start()
        for i in range(_N - _NSEM_TC, _N):
            cp(i).wait()

    return pl.pallas_call(
        tck,
        out_shape=jax.ShapeDtypeStruct((_CROP_L, _N, _D), jnp.float32),
        in_specs=[pl.BlockSpec(memory_space=pl.ANY)],
        out_specs=pl.BlockSpec(memory_space=pl.ANY),
        scratch_shapes=[pltpu.SemaphoreType.DMA] * _NSEM_TC,
    )(x2d)


def kernel(x):
    x2d = x.reshape(_N * _T, _D)
    out_a = _crop_copy(x2d)
    out_b = _dup_tc(x2d)
    s1 = jnp.transpose(out_a, (1, 0, 2))
    s2 = jnp.transpose(out_b, (1, 0, 2))
    left1 = jnp.asarray(_OFF + _CROP_ELEFT, dtype=jnp.int32)
    left2 = jnp.asarray(_START, dtype=jnp.int32)
    return (s1, left1, s2, left2, jnp.asarray(_CROP_L))


# speedup vs baseline: 18.3162x; 18.3162x over previous
import functools

import numpy as np
import jax
import jax.numpy as jnp
from jax import lax
from jax.experimental import pallas as pl
from jax.experimental.pallas import tpu as pltpu
from jax.experimental.pallas import tpu_sc as plsc

_N, _T, _D = 128, 2048, 128


def _crop_constants():
    np.random.seed(1)
    crop_l = int(np.random.randint(low=2, high=_T + 1))
    crop_left = int(np.random.randint(_T - crop_l + 1))
    crop_right = crop_left + crop_l
    crop_eleft = int(np.random.randint(crop_left + 1))
    crop_eright = int(np.random.randint(low=crop_right, high=_T + 1))
    crop_offset = np.random.randint(
        low=-crop_eleft, high=_T - crop_eright + 1, size=_N
    )
    return crop_l, crop_left, crop_eleft, crop_offset


_CROP_L, _CROP_LEFT, _CROP_ELEFT, _OFF = _crop_constants()
_START = _OFF + _CROP_LEFT
_B = _N * _CROP_L

_NC, _NS = 2, 16
_NW = _NC * _NS
_RPW = _N // _NW
_CB = 448

_BASES = np.zeros((_NW, _RPW), np.int64)
for _w in range(_NW):
    for _j in range(_RPW):
        _r = _w * _RPW + _j
        _BASES[_w, _j] = _r * _T + _START[_r]

_CHUNKS = []
for _j in range(_RPW):
    _o = 0
    while _o < _CROP_L:
        _ln = min(_CB, _CROP_L - _o)
        if _ln % 8:
            _ln8 = -(-_ln // 8) * 8
            _CHUNKS.append((_j, _CROP_L - _ln8, _ln8))
            break
        _CHUNKS.append((_j, _o, _ln))
        _o += _ln


def _crop_copy(x2d):
    mesh = plsc.VectorSubcoreMesh(core_axis_name="c", subcore_axis_name="s")
    out_sds = jax.ShapeDtypeStruct((_CROP_L, _N, _D), jnp.float32)

    @functools.partial(
        pl.kernel,
        mesh=mesh,
        out_type=out_sds,
        compiler_params=pltpu.CompilerParams(use_tc_tiling_on_sc=False),
        scratch_types=[
            pltpu.VMEM((_CB, _D), jnp.float32),
            pltpu.VMEM((_CB, _D), jnp.float32),
            pltpu.SemaphoreType.DMA,
            pltpu.SemaphoreType.DMA,
            pltpu.SemaphoreType.DMA,
            pltpu.SemaphoreType.DMA,
        ],
    )
    def k(x_hbm, out_a, buf0, buf1, sem0, sem1, wa0, wa1):
        wid = lax.axis_index("s") * _NC + lax.axis_index("c")
        src_base = []
        for j in range(_RPW):
            b = jnp.int32(int(_BASES[0, j]))
            for w in range(1, _NW):
                b = jnp.where(wid == w, jnp.int32(int(_BASES[w, j])), b)
            src_base.append(b)
        bufs = (buf0, buf1)
        sems = (sem0, sem1)
        wsems_a = (wa0, wa1)

        def src(t):
            j, o, ln = _CHUNKS[t]
            off = src_base[j] + o
            return x_hbm.at[pl.ds(off, ln)]

        def stage(t):
            j, o, ln = _CHUNKS[t]
            return bufs[t % 2].at[pl.ds(0, ln)]

        def dst(t, out_hbm):
            j, o, ln = _CHUNKS[t]
            rb = wid * _RPW + j
            return out_hbm.at[pl.ds(o, ln), rb]

        def wcopy_a(t):
            return pltpu.make_async_copy(
                stage(t), dst(t, out_a), wsems_a[t % 2]
            )

        n = len(_CHUNKS)
        pltpu.make_async_copy(src(0), stage(0), sems[0]).start()
        for t in range(n):
            b = t % 2
            pltpu.make_async_copy(src(t), stage(t), sems[b]).wait()
            if t + 1 < n:
                if t >= 1:
                    wcopy_a(t - 1).wait()
                pltpu.make_async_copy(
                    src(t + 1), stage(t + 1), sems[1 - b]
                ).start()
            wcopy_a(t).start()
        if n >= 2:
            wcopy_a(n - 2).wait()
        wcopy_a(n - 1).wait()

    return k(x2d)


_GRP = 8


def _dup_tc(x2d):
    starts = jnp.asarray(
        _START + np.arange(_N, dtype=np.int64) * _T, dtype=jnp.int32
    )

    def tck(starts_ref, x_hbm, out_ref, *sems):
        g = pl.program_id(0)

        def cp(j):
            base = starts_ref[g * _GRP + j]
            return pltpu.make_async_copy(
                x_hbm.at[pl.ds(base, _CROP_L)],
                out_ref.at[:, j, :],
                sems[j],
            )

        for j in range(_GRP):
            cp(j).start()
        for j in range(_GRP):
            cp(j).wait()

    return pl.pallas_call(
        tck,
        grid=(_N // _GRP,),
        in_specs=[
            pl.BlockSpec(memory_space=pltpu.MemorySpace.SMEM),
            pl.BlockSpec(memory_space=pl.ANY),
        ],
        out_specs=pl.BlockSpec((_CROP_L, _GRP, _D), lambda g: (0, g, 0)),
        out_shape=jax.ShapeDtypeStruct((_CROP_L, _N, _D), jnp.float32),
        scratch_shapes=[pltpu.SemaphoreType.DMA] * _GRP,
    )(starts, x2d)


def kernel(x):
    x2d = x.reshape(_N * _T, _D)
    out_a = _crop_copy(x2d)
    out_b = _dup_tc(x2d)
    s1 = jnp.transpose(out_a, (1, 0, 2))
    s2 = jnp.transpose(out_b, (1, 0, 2))
    left1 = jnp.asarray(_OFF + _CROP_ELEFT, dtype=jnp.int32)
    left2 = jnp.asarray(_START, dtype=jnp.int32)
    return (s1, left1, s2, left2, jnp.asarray(_CROP_L))


# speedup vs baseline: 21.4612x; 1.1717x over previous
import functools

import numpy as np
import jax
import jax.numpy as jnp
from jax import lax
from jax.experimental import pallas as pl
from jax.experimental.pallas import tpu as pltpu
from jax.experimental.pallas import tpu_sc as plsc

_N, _T, _D = 128, 2048, 128


def _crop_constants():
    np.random.seed(1)
    crop_l = int(np.random.randint(low=2, high=_T + 1))
    crop_left = int(np.random.randint(_T - crop_l + 1))
    crop_right = crop_left + crop_l
    crop_eleft = int(np.random.randint(crop_left + 1))
    crop_eright = int(np.random.randint(low=crop_right, high=_T + 1))
    crop_offset = np.random.randint(
        low=-crop_eleft, high=_T - crop_eright + 1, size=_N
    )
    return crop_l, crop_left, crop_eleft, crop_offset


_CROP_L, _CROP_LEFT, _CROP_ELEFT, _OFF = _crop_constants()
_START = _OFF + _CROP_LEFT
_B = _N * _CROP_L

_NC, _NS = 2, 16
_NW = _NC * _NS
_RPW = _N // _NW
_CB = 224

_BASES = np.zeros((_NW, _RPW), np.int64)
for _w in range(_NW):
    for _j in range(_RPW):
        _r = _w * _RPW + _j
        _BASES[_w, _j] = _r * _T + _START[_r]

_CHUNKS = []
for _j in range(_RPW):
    _o = 0
    while _o < _CROP_L:
        _ln = min(_CB, _CROP_L - _o)
        if _ln % 8:
            _ln8 = -(-_ln // 8) * 8
            _CHUNKS.append((_j, _CROP_L - _ln8, _ln8))
            break
        _CHUNKS.append((_j, _o, _ln))
        _o += _ln


def _crop_copy(x2d):
    mesh = plsc.VectorSubcoreMesh(core_axis_name="c", subcore_axis_name="s")
    out_sds = jax.ShapeDtypeStruct((_CROP_L, _N, _D), jnp.float32)

    @functools.partial(
        pl.kernel,
        mesh=mesh,
        out_type=(out_sds, out_sds),
        compiler_params=pltpu.CompilerParams(use_tc_tiling_on_sc=False),
        scratch_types=[
            pltpu.VMEM((_CB, _D), jnp.float32),
            pltpu.VMEM((_CB, _D), jnp.float32),
            pltpu.SemaphoreType.DMA,
            pltpu.SemaphoreType.DMA,
        ],
    )
    def k(x_hbm, out_a, out_b, buf0, buf1, sem0, sem1):
        wid = lax.axis_index("s") * _NC + lax.axis_index("c")
        src_base = []
        for j in range(_RPW):
            b = jnp.int32(int(_BASES[0, j]))
            for w in range(1, _NW):
                b = jnp.where(wid == w, jnp.int32(int(_BASES[w, j])), b)
            src_base.append(b)
        bufs = (buf0, buf1)
        sems = (sem0, sem1)

        def src(t):
            j, o, ln = _CHUNKS[t]
            off = src_base[j] + o
            return x_hbm.at[pl.ds(off, ln)]

        def stage(t):
            j, o, ln = _CHUNKS[t]
            return bufs[t % 2].at[pl.ds(0, ln)]

        def dst(t, out_hbm):
            j, o, ln = _CHUNKS[t]
            rb = wid * _RPW + j
            return out_hbm.at[pl.ds(o, ln), rb]

        pltpu.make_async_copy(src(0), stage(0), sems[0]).start()
        for t in range(len(_CHUNKS)):
            b = t % 2
            pltpu.make_async_copy(src(t), stage(t), sems[b]).wait()
            if t + 1 < len(_CHUNKS):
                pltpu.make_async_copy(
                    src(t + 1), stage(t + 1), sems[1 - b]
                ).start()
            pltpu.sync_copy(stage(t), dst(t, out_a))
            pltpu.sync_copy(stage(t), dst(t, out_b))

    return k(x2d)


def kernel(x):
    x2d = x.reshape(_N * _T, _D)
    out_a, out_b = _crop_copy(x2d)
    s1 = jnp.transpose(out_a, (1, 0, 2))
    s2 = jnp.transpose(out_b, (1, 0, 2))
    left1 = jnp.asarray(_OFF + _CROP_ELEFT, dtype=jnp.int32)
    left2 = jnp.asarray(_START, dtype=jnp.int32)
    return (s1, left1, s2, left2, jnp.asarray(_CROP_L))


# speedup vs baseline: 22.1381x; 1.0315x over previous
import functools

import numpy as np
import jax
import jax.numpy as jnp
from jax import lax
from jax.experimental import pallas as pl
from jax.experimental.pallas import tpu as pltpu
from jax.experimental.pallas import tpu_sc as plsc

_N, _T, _D = 128, 2048, 128


def _crop_constants():
    np.random.seed(1)
    crop_l = int(np.random.randint(low=2, high=_T + 1))
    crop_left = int(np.random.randint(_T - crop_l + 1))
    crop_right = crop_left + crop_l
    crop_eleft = int(np.random.randint(crop_left + 1))
    crop_eright = int(np.random.randint(low=crop_right, high=_T + 1))
    crop_offset = np.random.randint(
        low=-crop_eleft, high=_T - crop_eright + 1, size=_N
    )
    return crop_l, crop_left, crop_eleft, crop_offset


_CROP_L, _CROP_LEFT, _CROP_ELEFT, _OFF = _crop_constants()
_START = _OFF + _CROP_LEFT
_B = _N * _CROP_L

_NC, _NS = 2, 16
_NW = _NC * _NS
_RPW = _N // _NW
_CB = 448

_BASES = np.zeros((_NW, _RPW), np.int64)
for _w in range(_NW):
    for _j in range(_RPW):
        _r = _w * _RPW + _j
        _BASES[_w, _j] = _r * _T + _START[_r]

_CHUNKS = []
for _j in range(_RPW):
    _o = 0
    while _o < _CROP_L:
        _ln = min(_CB, _CROP_L - _o)
        if _ln % 8:
            _ln8 = -(-_ln // 8) * 8
            _CHUNKS.append((_j, _CROP_L - _ln8, _ln8))
            break
        _CHUNKS.append((_j, _o, _ln))
        _o += _ln


def _crop_copy(x2d):
    mesh = plsc.VectorSubcoreMesh(core_axis_name="c", subcore_axis_name="s")
    out_sds = jax.ShapeDtypeStruct((_CROP_L, _N, _D), jnp.float32)

    @functools.partial(
        pl.kernel,
        mesh=mesh,
        out_type=(out_sds, out_sds),
        compiler_params=pltpu.CompilerParams(use_tc_tiling_on_sc=False),
        scratch_types=[
            pltpu.VMEM((_CB, _D), jnp.float32),
            pltpu.VMEM((_CB, _D), jnp.float32),
            pltpu.SemaphoreType.DMA,
            pltpu.SemaphoreType.DMA,
        ],
    )
    def k(x_hbm, out_a, out_b, buf0, buf1, sem0, sem1):
        wid = lax.axis_index("s") * _NC + lax.axis_index("c")
        src_base = []
        for j in range(_RPW):
            b = jnp.int32(int(_BASES[0, j]))
            for w in range(1, _NW):
                b = jnp.where(wid == w, jnp.int32(int(_BASES[w, j])), b)
            src_base.append(b)
        bufs = (buf0, buf1)
        sems = (sem0, sem1)

        def src(t):
            j, o, ln = _CHUNKS[t]
            off = src_base[j] + o
            return x_hbm.at[pl.ds(off, ln)]

        def stage(t):
            j, o, ln = _CHUNKS[t]
            return bufs[t % 2].at[pl.ds(0, ln)]

        def dst(t, out_hbm):
            j, o, ln = _CHUNKS[t]
            rb = wid * _RPW + j
            return out_hbm.at[pl.ds(o, ln), rb]

        pltpu.make_async_copy(src(0), stage(0), sems[0]).start()
        for t in range(len(_CHUNKS)):
            b = t % 2
            pltpu.make_async_copy(src(t), stage(t), sems[b]).wait()
            if t + 1 < len(_CHUNKS):
                pltpu.make_async_copy(
                    src(t + 1), stage(t + 1), sems[1 - b]
                ).start()
            pltpu.sync_copy(stage(t), dst(t, out_a))
            pltpu.sync_copy(stage(t), dst(t, out_b))

    return k(x2d)


def kernel(x):
    x2d = x.reshape(_N * _T, _D)
    out_a, out_b = _crop_copy(x2d)
    s1 = jnp.transpose(out_a, (1, 0, 2))
    s2 = jnp.transpose(out_b, (1, 0, 2))
    left1 = jnp.asarray(_OFF + _CROP_ELEFT, dtype=jnp.int32)
    left2 = jnp.asarray(_START, dtype=jnp.int32)
    return (s1, left1, s2, left2, jnp.asarray(_CROP_L))
